# Initial kernel scaffold; baseline (speedup 1.0000x reference)
import functools

import jax
import jax.numpy as jnp
from jax import lax
from jax.experimental import pallas as pl
from jax.experimental.pallas import tpu as pltpu
from jax.experimental.pallas import tpu_sc as plsc

N = 10000
E = 160000
F_IN = 128
HID = 256
NC = 40

NCORES = 2
NSUB = 16
EB = E // NSUB
RPT = N // NSUB
ABATCH = 2000
G = 400

_mesh = plsc.VectorSubcoreMesh(core_axis_name="c", subcore_axis_name="s")


def _mm_chunks_body(x_ref, w_ref, o_ref):
    o_ref[0] = jnp.dot(x_ref[...], w_ref[...],
                       preferred_element_type=jnp.float32)


def _mm_chunks(x, w, chunks, cw, bn=1000):
    f = x.shape[1]
    nb = N // bn
    return pl.pallas_call(
        _mm_chunks_body,
        grid=(nb, chunks),
        in_specs=[pl.BlockSpec((bn, f), lambda i, c: (i, 0)),
                  pl.BlockSpec((f, cw), lambda i, c: (0, c))],
        out_specs=pl.BlockSpec((1, bn, cw), lambda i, c: (c, i, 0)),
        out_shape=jax.ShapeDtypeStruct((chunks, N, cw), jnp.float32),
    )(x, w)


def _attn_body(x_ref, w_ref, ams_ref, amd_ref, oas_ref, oad_ref):
    x = x_ref[...]
    was = jnp.dot(w_ref[...], ams_ref[...], preferred_element_type=jnp.float32)
    wad = jnp.dot(w_ref[...], amd_ref[...], preferred_element_type=jnp.float32)
    oas_ref[0] = jnp.dot(x, was, preferred_element_type=jnp.float32)
    oad_ref[0] = jnp.dot(x, wad, preferred_element_type=jnp.float32)


def _attn(x, w, amat_s, amat_d, hd, c, bn=1000):
    f = x.shape[1]
    hdh = hd // 2
    nb = N // bn
    out = jax.ShapeDtypeStruct((2, N, hdh), jnp.float32)
    return pl.pallas_call(
        _attn_body,
        grid=(nb, 2),
        in_specs=[pl.BlockSpec((bn, f), lambda i, s: (i, 0)),
                  pl.BlockSpec((f, hdh * c), lambda i, s: (0, s)),
                  pl.BlockSpec((hdh * c, hdh), lambda i, s: (s, s)),
                  pl.BlockSpec((hdh * c, hdh), lambda i, s: (s, s))],
        out_specs=[pl.BlockSpec((1, bn, hdh), lambda i, s: (s, i, 0)),
                   pl.BlockSpec((1, bn, hdh), lambda i, s: (s, i, 0))],
        out_shape=[out, out],
    )(x, w, amat_s, amat_d)


def _combine_body(m_ref, x_ref, lw_ref, b_ref, lb_ref, o_ref):
    r = (m_ref[0] + b_ref[...][None, :] + lb_ref[...][None, :]
         + jnp.dot(x_ref[...], lw_ref[...], preferred_element_type=jnp.float32))
    o_ref[...] = jnp.where(r > 0, r, jnp.expm1(r))


def _combine(msg, x, lw, b, lb, chunks, cw, bn=1000):
    f = x.shape[1]
    nb = N // bn
    return pl.pallas_call(
        _combine_body,
        grid=(nb, chunks),
        in_specs=[pl.BlockSpec((1, bn, cw), lambda i, c: (c, i, 0)),
                  pl.BlockSpec((bn, f), lambda i, c: (i, 0)),
                  pl.BlockSpec((f, cw), lambda i, c: (0, c)),
                  pl.BlockSpec((cw,), lambda i, c: (c,)),
                  pl.BlockSpec((cw,), lambda i, c: (c,))],
        out_specs=pl.BlockSpec((bn, cw), lambda i, c: (i, c)),
        out_shape=jax.ShapeDtypeStruct((N, chunks * cw), jnp.float32),
    )(msg.reshape(chunks, N, cw), x, lw, b, lb)


def _final_body(m_ref, x_ref, lw_ref, b_ref, lb_ref, o_ref):
    m = jnp.sum(m_ref[...], axis=0) * (1.0 / 6.0)
    o_ref[...] = (m + b_ref[...][None, :] + lb_ref[...][None, :]
                  + jnp.dot(x_ref[...], lw_ref[...],
                            preferred_element_type=jnp.float32))


def _final(msg, x, lw, b, lb, bn=1000):
    f = x.shape[1]
    nb = N // bn
    return pl.pallas_call(
        _final_body,
        grid=(nb,),
        in_specs=[pl.BlockSpec((6, bn, 64), lambda i: (0, i, 0)),
                  pl.BlockSpec((bn, f), lambda i: (i, 0)),
                  pl.BlockSpec((f, 64), lambda i: (0, 0)),
                  pl.BlockSpec((64,), lambda i: (0,)),
                  pl.BlockSpec((64,), lambda i: (0,))],
        out_specs=pl.BlockSpec((bn, 64), lambda i: (i, 0)),
        out_shape=jax.ShapeDtypeStruct((N, 64), jnp.float32),
    )(msg.reshape(6, N, 64), x, lw, b, lb)


def _alpha_body(src_hbm, dst_hbm, as_hbm, ad_hbm, alpha_hbm,
                asv, adv, denv, sv, dv, exv, stage, den_sp, *, hdh):
    cax = lax.axis_index("c")
    tid = lax.axis_index("s")
    e0 = tid * EB
    nbatch = EB // ABATCH

    pltpu.sync_copy(as_hbm.at[cax], asv)
    pltpu.sync_copy(ad_hbm.at[cax], adv)
    pltpu.sync_copy(src_hbm.at[pl.ds(e0, EB)], sv)
    for b in range(nbatch):
        pltpu.sync_copy(dst_hbm.at[pl.ds(e0 + b * ABATCH, ABATCH)], dv.at[b])

    zero16 = jnp.zeros((16,), jnp.float32)
    for h in range(hdh):
        hsplat = jnp.full((16,), h, jnp.int32)

        def zb(i, _, hsplat=hsplat):
            plsc.store_scatter(exv, [lax.iota(jnp.int32, 16) + i * 16, hsplat],
                               zero16)
            return 0
        lax.fori_loop(0, ABATCH // 16, zb, 0)
    pltpu.sync_copy(exv.at[pl.ds(0, RPT)], den_sp.at[pl.ds(tid * RPT, RPT)])
    plsc.subcore_barrier()

    def _ex16(b, i, hsplat):
        s16 = sv[pl.ds(b * ABATCH + i * 16, 16)]
        d16 = dv[b, pl.ds(i * 16, 16)]
        av = plsc.load_gather(asv, [s16, hsplat])
        bv = plsc.load_gather(adv, [d16, hsplat])
        e16 = av + bv
        e16 = jnp.where(e16 >= 0, e16, 0.2 * e16)
        return d16, jnp.exp(e16)

    for b in range(nbatch):
        for h in range(hdh):
            hsplat = jnp.full((16,), h, jnp.int32)

            def p1(i, _, b=b, hsplat=hsplat):
                _, ex16 = _ex16(b, i, hsplat)
                plsc.store_scatter(
                    exv, [lax.iota(jnp.int32, 16) + i * 16, hsplat], ex16)
                return 0
            lax.fori_loop(0, ABATCH // 16, p1, 0)
        pltpu.sync_copy(exv, den_sp.at[dv.at[b]], add=True)
    plsc.subcore_barrier()
    pltpu.sync_copy(den_sp, denv)

    for b in range(nbatch):
        for h in range(hdh):
            hsplat = jnp.full((16,), h, jnp.int32)

            def p2(i, _, b=b, hsplat=hsplat):
                d16, ex16 = _ex16(b, i, hsplat)
                den16 = plsc.load_gather(denv, [d16, hsplat])
                stage[pl.ds(i * 16, 16)] = ex16 / (den16 + 1e-16)
                return 0
            lax.fori_loop(0, ABATCH // 16, p2, 0)
            pltpu.sync_copy(
                stage,
                alpha_hbm.at[cax * hdh + h, pl.ds(e0 + b * ABATCH, ABATCH)])


def _alpha(src, dst, ass, ads, hd):
    hdh = hd // 2
    body = functools.partial(_alpha_body, hdh=hdh)
    return pl.kernel(
        body,
        out_type=jax.ShapeDtypeStruct((hd, E), jnp.float32),
        mesh=_mesh,
        scratch_types=[
            pltpu.VMEM((N, hdh), jnp.float32),
            pltpu.VMEM((N, hdh), jnp.float32),
            pltpu.VMEM((N, hdh), jnp.float32),
            pltpu.VMEM((EB,), jnp.int32),
            pltpu.VMEM((EB // ABATCH, ABATCH), jnp.int32),
            pltpu.VMEM((ABATCH, hdh), jnp.float32),
            pltpu.VMEM((ABATCH,), jnp.float32),
            pltpu.MemorySpace.VMEM_SHARED((N, hdh), jnp.float32),
        ],
    )(src, dst, ass, ads)


def _msg_body(src_hbm, dst_hbm, alpha_hbm, h_hbm, out_hbm,
              sv, dv, svo, dvo, av, rows, zrows, sem, acc_sp,
              *, ch, cw, hpc):
    cax = lax.axis_index("c")
    tid = lax.axis_index("s")
    e0 = tid * EB
    chh = ch // 2
    nv = cw // 16

    pltpu.sync_copy(src_hbm.at[pl.ds(e0, EB)], sv)
    pltpu.sync_copy(dst_hbm.at[pl.ds(e0, EB)], dv)

    zero16 = jnp.zeros((16,), jnp.float32)

    def zb(i, _):
        r = i // nv
        v = i % nv
        zrows[r, pl.ds(v * 16, 16)] = zero16
        return 0
    lax.fori_loop(0, (RPT // 5) * nv, zb, 0)

    for k in range(chh):
        c = cax * chh + k
        head = c // hpc
        for z in range(5):
            pltpu.sync_copy(
                zrows, acc_sp.at[pl.ds(tid * RPT + z * (RPT // 5), RPT // 5)])
        plsc.subcore_barrier()

        pltpu.sync_copy(alpha_hbm.at[head, pl.ds(e0, EB)], av)

        def gbody(g, _, c=c):
            lg = g * G

            def obody(i, _):
                svo[pl.ds(i * 16, 16)] = sv[pl.ds(lg + i * 16, 16)] + c * N
                dvo[pl.ds(i * 16, 16)] = dv[pl.ds(lg + i * 16, 16)]
                return 0
            lax.fori_loop(0, G // 16, obody, 0)

            pltpu.async_copy(h_hbm.at[svo], rows, sem).wait()

            def sbody(j, _):
                a = av[lg + j]
                for v in range(nv):
                    rows[j, pl.ds(v * 16, 16)] = rows[j, pl.ds(v * 16, 16)] * a
                return 0
            lax.fori_loop(0, G, sbody, 0)

            pltpu.sync_copy(rows, acc_sp.at[dvo], add=True)
            return 0
        lax.fori_loop(0, EB // G, gbody, 0)
        plsc.subcore_barrier()

        pltpu.sync_copy(acc_sp.at[pl.ds(tid * RPT, RPT)],
                        out_hbm.at[pl.ds(c * N + tid * RPT, RPT)])
        plsc.subcore_barrier()


def _msg(src, dst, alpha_t, hflat, ch, cw, hpc):
    body = functools.partial(_msg_body, ch=ch, cw=cw, hpc=hpc)
    return pl.kernel(
        body,
        out_type=jax.ShapeDtypeStruct((ch * N, cw), jnp.float32),
        mesh=_mesh,
        scratch_types=[
            pltpu.VMEM((EB,), jnp.int32),
            pltpu.VMEM((EB,), jnp.int32),
            pltpu.VMEM((G,), jnp.int32),
            pltpu.VMEM((G,), jnp.int32),
            pltpu.VMEM((EB,), jnp.float32),
            pltpu.VMEM((G, cw), jnp.float32),
            pltpu.VMEM((RPT // 5, cw), jnp.float32),
            pltpu.SemaphoreType.DMA,
            pltpu.MemorySpace.VMEM_SHARED((N, cw), jnp.float32),
        ],
    )(src, dst, alpha_t, hflat)


def _block_diag_a(a, c):
    hd = a.shape[0]
    mask = jnp.repeat(jnp.eye(hd, dtype=jnp.float32), c, axis=0)
    return mask * a.reshape(hd * c, 1)


def _gat_layer(x, src, dst, w, amat_s, amat_d, hd, chunks, cw, hpc):
    h = _mm_chunks(x, w, chunks, cw)
    ass, ads = _attn(x, w, amat_s, amat_d, hd, (chunks * cw) // hd)
    alpha_t = _alpha(src, dst, ass, ads, hd)
    msg = _msg(src, dst, alpha_t, h.reshape(chunks * N, cw), chunks, cw, hpc)
    return msg, alpha_t


def kernel(x, edge_index, W1, a1s, a1d, b1, L1W, L1b,
           W2, a2s, a2d, b2, L2W, L2b, W3, a3s, a3d, b3, L3W, L3b):
    src = edge_index[0]
    dst = edge_index[1]

    m1, alpha1_t = _gat_layer(x, src, dst, W1,
                              _block_diag_a(a1s, HID), _block_diag_a(a1d, HID),
                              hd=4, chunks=8, cw=128, hpc=2)
    x1 = _combine(m1, x, L1W, b1, L1b, chunks=8, cw=128)

    m2, _ = _gat_layer(x1, src, dst, W2,
                       _block_diag_a(a2s, HID), _block_diag_a(a2d, HID),
                       hd=4, chunks=8, cw=128, hpc=2)
    x2 = _combine(m2, x1, L2W, b2, L2b, chunks=8, cw=128)

    w3p = jnp.pad(W3.reshape(4 * HID, 6, NC), ((0, 0), (0, 0), (0, 64 - NC))
                  ).reshape(4 * HID, 6 * 64)
    a3sp = jnp.pad(a3s, ((0, 0), (0, 64 - NC)))
    a3dp = jnp.pad(a3d, ((0, 0), (0, 64 - NC)))
    l3wp = jnp.pad(L3W, ((0, 0), (0, 64 - NC)))
    b3p = jnp.pad(b3, (0, 64 - NC))
    l3bp = jnp.pad(L3b, (0, 64 - NC))
    m3, _ = _gat_layer(x2, src, dst, w3p,
                       _block_diag_a(a3sp, 64), _block_diag_a(a3dp, 64),
                       hd=6, chunks=6, cw=64, hpc=1)
    outp = _final(m3, x2, l3wp, b3p, l3bp)

    return outp[:, :NC], alpha1_t.T


# speedup vs baseline: 16.7258x; 16.7258x over previous
import functools

import jax
import jax.numpy as jnp
from jax import lax
from jax.experimental import pallas as pl
from jax.experimental.pallas import tpu as pltpu
from jax.experimental.pallas import tpu_sc as plsc

N = 10000
E = 160000
F_IN = 128
HID = 256
NC = 40

NCORES = 2
NSUB = 16
EB = E // NSUB
NP = 10240
WR = NP // NSUB
ABATCH = 400
G = 80

_mesh = plsc.VectorSubcoreMesh(core_axis_name="c", subcore_axis_name="s",
                               num_cores=NCORES, num_subcores=NSUB)


def _mm_chunks_body(x_ref, w_ref, o_ref):
    o_ref[0] = jnp.dot(x_ref[...], w_ref[0],
                       preferred_element_type=jnp.float32)


def _mm_chunks(x, wc, chunks, cw, bn=1000):
    f = x.shape[1]
    nb = N // bn
    return pl.pallas_call(
        _mm_chunks_body,
        grid=(nb, chunks),
        in_specs=[pl.BlockSpec((bn, f), lambda i, c: (i, 0)),
                  pl.BlockSpec((1, f, cw), lambda i, c: (c, 0, 0))],
        out_specs=pl.BlockSpec((1, bn, cw), lambda i, c: (c, i, 0)),
        out_shape=jax.ShapeDtypeStruct((chunks, N, cw), jnp.float32),
    )(x, wc)


def _attn_body(x_ref, w_ref, ams_ref, amd_ref, oas_ref, oad_ref):
    x = x_ref[...]
    was = jnp.dot(w_ref[0], ams_ref[0], preferred_element_type=jnp.float32)
    wad = jnp.dot(w_ref[0], amd_ref[0], preferred_element_type=jnp.float32)
    oas_ref[0] = jnp.dot(x, was, preferred_element_type=jnp.float32)
    oad_ref[0] = jnp.dot(x, wad, preferred_element_type=jnp.float32)


def _attn(x, wh, amh_s, amh_d, hd, c, bn=1000):
    f = x.shape[1]
    hdh = hd // 2
    nb = N // bn
    out = jax.ShapeDtypeStruct((2, N, hdh), jnp.float32)
    return pl.pallas_call(
        _attn_body,
        grid=(nb, 2),
        in_specs=[pl.BlockSpec((bn, f), lambda i, s: (i, 0)),
                  pl.BlockSpec((1, f, hdh * c), lambda i, s: (s, 0, 0)),
                  pl.BlockSpec((1, hdh * c, hdh), lambda i, s: (s, 0, 0)),
                  pl.BlockSpec((1, hdh * c, hdh), lambda i, s: (s, 0, 0))],
        out_specs=[pl.BlockSpec((1, bn, hdh), lambda i, s: (s, i, 0)),
                   pl.BlockSpec((1, bn, hdh), lambda i, s: (s, i, 0))],
        out_shape=[out, out],
    )(x, wh, amh_s, amh_d)


def _combine_body(m_ref, x_ref, lw_ref, b_ref, lb_ref, o_ref):
    r = (m_ref[0] + b_ref[...][None, :] + lb_ref[...][None, :]
         + jnp.dot(x_ref[...], lw_ref[...], preferred_element_type=jnp.float32))
    o_ref[...] = jnp.where(r > 0, r, jnp.exp(r) - 1.0)


def _combine(msg, x, lw, b, lb, chunks, cw, bn=1000):
    f = x.shape[1]
    nb = N // bn
    return pl.pallas_call(
        _combine_body,
        grid=(nb, chunks),
        in_specs=[pl.BlockSpec((1, bn, cw), lambda i, c: (c, i, 0)),
                  pl.BlockSpec((bn, f), lambda i, c: (i, 0)),
                  pl.BlockSpec((f, cw), lambda i, c: (0, c)),
                  pl.BlockSpec((cw,), lambda i, c: (c,)),
                  pl.BlockSpec((cw,), lambda i, c: (c,))],
        out_specs=pl.BlockSpec((bn, cw), lambda i, c: (i, c)),
        out_shape=jax.ShapeDtypeStruct((N, chunks * cw), jnp.float32),
    )(msg.reshape(chunks, NP, cw), x, lw, b, lb)


def _final_body(m_ref, x_ref, lw_ref, b_ref, lb_ref, o_ref):
    m = jnp.sum(m_ref[...], axis=0) * (1.0 / 6.0)
    o_ref[...] = (m + b_ref[...][None, :] + lb_ref[...][None, :]
                  + jnp.dot(x_ref[...], lw_ref[...],
                            preferred_element_type=jnp.float32))


def _final(msg, x, lw, b, lb, bn=1000):
    f = x.shape[1]
    nb = N // bn
    return pl.pallas_call(
        _final_body,
        grid=(nb,),
        in_specs=[pl.BlockSpec((6, bn, 64), lambda i: (0, i, 0)),
                  pl.BlockSpec((bn, f), lambda i: (i, 0)),
                  pl.BlockSpec((f, 64), lambda i: (0, 0)),
                  pl.BlockSpec((64,), lambda i: (0,)),
                  pl.BlockSpec((64,), lambda i: (0,))],
        out_specs=pl.BlockSpec((bn, 64), lambda i: (i, 0)),
        out_shape=jax.ShapeDtypeStruct((N, 64), jnp.float32),
    )(msg.reshape(6, NP, 64), x, lw, b, lb)


def _alpha_body(src_hbm, dst_hbm, as_hbm, ad_hbm, alpha_hbm,
                asv, adv, sv, dv, dvo, exv, denr, stage, den_sp, *, hdh):
    cax = lax.axis_index("c")
    tid = lax.axis_index("s")
    e0 = tid * EB
    nbatch = EB // ABATCH
    nh = N * hdh

    pltpu.sync_copy(as_hbm.at[pl.ds(cax * nh, nh)], asv)
    pltpu.sync_copy(ad_hbm.at[pl.ds(cax * nh, nh)], adv)
    pltpu.sync_copy(src_hbm.at[pl.ds(e0, EB)], sv)
    pltpu.sync_copy(dst_hbm.at[pl.ds(e0, EB)], dv)

    zero16 = jnp.zeros((16,), jnp.float32)

    def zb(i, _):
        exv[i, pl.ds(0, 16)] = zero16
        return 0
    lax.fori_loop(0, ABATCH, zb, 0)
    pltpu.sync_copy(exv, den_sp.at[pl.ds(tid * WR, ABATCH)])
    pltpu.sync_copy(exv.at[pl.ds(0, WR - ABATCH)],
                    den_sp.at[pl.ds(tid * WR + ABATCH, WR - ABATCH)])
    plsc.subcore_barrier()

    def _ex16(b, i, hsplat):
        j = b * ABATCH + i * 16
        s16 = sv[pl.ds(j, 16)]
        d16 = dv[pl.ds(j, 16)]
        av = plsc.load_gather(asv, [s16 * hdh + hsplat])
        bv = plsc.load_gather(adv, [d16 * hdh + hsplat])
        e16 = av + bv
        e16 = jnp.where(e16 >= 0, e16, 0.2 * e16)
        return jnp.exp(e16)

    def fill_dvo(b):
        def ob(i, _):
            dvo[pl.ds(i * 16, 16)] = dv[pl.ds(b * ABATCH + i * 16, 16)]
            return 0
        lax.fori_loop(0, ABATCH // 16, ob, 0)

    def p1b(b, _):
        fill_dvo(b)
        for h in range(hdh):
            hsplat = jnp.full((16,), h, jnp.int32)

            def p1(i, _, b=b, hsplat=hsplat):
                ex16 = _ex16(b, i, hsplat)
                plsc.store_scatter(
                    exv, [lax.iota(jnp.int32, 16) + i * 16, hsplat], ex16)
                return 0
            lax.fori_loop(0, ABATCH // 16, p1, 0)
        pltpu.sync_copy(exv, den_sp.at[dvo], add=True)
        return 0
    lax.fori_loop(0, nbatch, p1b, 0)
    plsc.subcore_barrier()

    def p2b(b, _):
        fill_dvo(b)
        pltpu.sync_copy(den_sp.at[dvo], denr)
        for h in range(hdh):
            hsplat = jnp.full((16,), h, jnp.int32)

            def p2(i, _, b=b, hsplat=hsplat):
                ex16 = _ex16(b, i, hsplat)
                e16i = lax.iota(jnp.int32, 16) + i * 16
                den16 = plsc.load_gather(denr, [e16i, hsplat])
                stage[pl.ds(i * 16, 16)] = ex16 / (den16 + 1e-16)
                return 0
            lax.fori_loop(0, ABATCH // 16, p2, 0)
            pltpu.sync_copy(
                stage,
                alpha_hbm.at[pl.ds((cax * hdh + h) * E + e0 + b * ABATCH,
                                   ABATCH)])
        return 0
    lax.fori_loop(0, nbatch, p2b, 0)


def _alpha(src, dst, ass, ads, hd):
    hdh = hd // 2
    body = functools.partial(_alpha_body, hdh=hdh)
    return pl.kernel(
        body,
        out_type=jax.ShapeDtypeStruct((hd * E,), jnp.float32),
        mesh=_mesh,
        compiler_params=pltpu.CompilerParams(
            needs_layout_passes=False, use_tc_tiling_on_sc=False),
        scratch_types=[
            pltpu.VMEM((N * hdh,), jnp.float32),
            pltpu.VMEM((N * hdh,), jnp.float32),
            pltpu.VMEM((EB,), jnp.int32),
            pltpu.VMEM((EB,), jnp.int32),
            pltpu.VMEM((ABATCH,), jnp.int32),
            pltpu.VMEM((ABATCH, 16), jnp.float32),
            pltpu.VMEM((ABATCH, 16), jnp.float32),
            pltpu.VMEM((ABATCH,), jnp.float32),
            pltpu.MemorySpace.VMEM_SHARED((NP, 16), jnp.float32),
        ],
    )(src, dst, ass, ads)


def _msg_body(src_hbm, dst_hbm, alpha_hbm, h_hbm, out_hbm,
              sv, dv, svo, dvo, av, rows, sem, acc_sp,
              *, ch, cw, hpc):
    cax = lax.axis_index("c")
    tid = lax.axis_index("s")
    e0 = tid * EB
    chh = ch // 2
    nv = cw // 16

    pltpu.sync_copy(src_hbm.at[pl.ds(e0, EB)], sv)
    pltpu.sync_copy(dst_hbm.at[pl.ds(e0, EB)], dv)

    zero16 = jnp.zeros((16,), jnp.float32)

    for k in range(chh):
        c = cax * chh + k
        head = c // hpc

        def zb(i, _):
            r = i // nv
            v = i % nv
            rows[r, pl.ds(v * 16, 16)] = zero16
            return 0
        lax.fori_loop(0, G * nv, zb, 0)
        for z in range(WR // G):
            pltpu.sync_copy(rows, acc_sp.at[pl.ds(tid * WR + z * G, G)])
        plsc.subcore_barrier()

        pltpu.sync_copy(alpha_hbm.at[pl.ds(head * E + e0, EB)], av)

        def gbody(g, _, c=c):
            lg = g * G

            def obody(i, _):
                svo[pl.ds(i * 16, 16)] = sv[pl.ds(lg + i * 16, 16)] + c * N
                dvo[pl.ds(i * 16, 16)] = dv[pl.ds(lg + i * 16, 16)]
                return 0
            lax.fori_loop(0, G // 16, obody, 0)

            pltpu.async_copy(h_hbm.at[svo], rows, sem).wait()

            def sbody(q, _):
                a16 = av[pl.ds(lg + q * 16, 16)]
                for u in range(16):
                    j = q * 16 + u
                    a = a16[u]
                    for v in range(nv):
                        rows[j, pl.ds(v * 16, 16)] = (
                            rows[j, pl.ds(v * 16, 16)] * a)
                return 0
            lax.fori_loop(0, G // 16, sbody, 0)

            pltpu.sync_copy(rows, acc_sp.at[dvo], add=True)
            return 0
        lax.fori_loop(0, EB // G, gbody, 0)
        plsc.subcore_barrier()

        pltpu.sync_copy(acc_sp.at[pl.ds(tid * WR, WR)],
                        out_hbm.at[pl.ds(c * NP + tid * WR, WR)])
        plsc.subcore_barrier()


def _msg(src, dst, alpha_t, hflat, ch, cw, hpc):
    body = functools.partial(_msg_body, ch=ch, cw=cw, hpc=hpc)
    return pl.kernel(
        body,
        out_type=jax.ShapeDtypeStruct((ch * NP, cw), jnp.float32),
        mesh=_mesh,
        compiler_params=pltpu.CompilerParams(
            needs_layout_passes=False, use_tc_tiling_on_sc=False),
        scratch_types=[
            pltpu.VMEM((EB,), jnp.int32),
            pltpu.VMEM((EB,), jnp.int32),
            pltpu.VMEM((G,), jnp.int32),
            pltpu.VMEM((G,), jnp.int32),
            pltpu.VMEM((EB,), jnp.float32),
            pltpu.VMEM((G, cw), jnp.float32),
            pltpu.SemaphoreType.DMA,
            pltpu.MemorySpace.VMEM_SHARED((NP, cw), jnp.float32),
        ],
    )(src, dst, alpha_t, hflat)


def _block_diag_halves(a, c):
    hd = a.shape[0]
    hdh = hd // 2
    mask = jnp.repeat(jnp.eye(hdh, dtype=jnp.float32), c, axis=0)
    halves = [mask * a[s * hdh:(s + 1) * hdh].reshape(hdh * c, 1)
              for s in range(2)]
    return jnp.stack(halves)


def _gat_layer(x, src, dst, w, a_s, a_d, hd, chunks, cw, hpc):
    f = x.shape[1]
    c = (chunks * cw) // hd
    wc = w.reshape(f, chunks, cw).transpose(1, 0, 2)
    wh = w.reshape(f, 2, (hd // 2) * c).transpose(1, 0, 2)
    h = _mm_chunks(x, wc, chunks, cw)
    ass, ads = _attn(x, wh, _block_diag_halves(a_s, c),
                     _block_diag_halves(a_d, c), hd, c)
    alpha_t = _alpha(src, dst, ass.reshape(2 * N * (hd // 2)),
                     ads.reshape(2 * N * (hd // 2)), hd)
    msg = _msg(src, dst, alpha_t, h.reshape(chunks * N, cw), chunks, cw, hpc)
    return msg, alpha_t


def kernel(x, edge_index, W1, a1s, a1d, b1, L1W, L1b,
           W2, a2s, a2d, b2, L2W, L2b, W3, a3s, a3d, b3, L3W, L3b):
    src = edge_index[0]
    dst = edge_index[1]

    m1, alpha1_t = _gat_layer(x, src, dst, W1, a1s, a1d,
                              hd=4, chunks=8, cw=128, hpc=2)
    x1 = _combine(m1, x, L1W, b1, L1b, chunks=8, cw=128)

    m2, _ = _gat_layer(x1, src, dst, W2, a2s, a2d,
                       hd=4, chunks=8, cw=128, hpc=2)
    x2 = _combine(m2, x1, L2W, b2, L2b, chunks=8, cw=128)

    w3p = jnp.pad(W3.reshape(4 * HID, 6, NC), ((0, 0), (0, 0), (0, 64 - NC))
                  ).reshape(4 * HID, 6 * 64)
    a3sp = jnp.pad(a3s, ((0, 0), (0, 64 - NC)))
    a3dp = jnp.pad(a3d, ((0, 0), (0, 64 - NC)))
    l3wp = jnp.pad(L3W, ((0, 0), (0, 64 - NC)))
    b3p = jnp.pad(b3, (0, 64 - NC))
    l3bp = jnp.pad(L3b, (0, 64 - NC))
    m3, _ = _gat_layer(x2, src, dst, w3p, a3sp, a3dp,
                       hd=6, chunks=6, cw=64, hpc=1)
    outp = _final(m3, x2, l3wp, b3p, l3bp)

    return outp[:, :NC], alpha1_t.reshape(4, E).T


# speedup vs baseline: 24.7895x; 1.4821x over previous
import functools

import jax
import jax.numpy as jnp
from jax import lax
from jax.experimental import pallas as pl
from jax.experimental.pallas import tpu as pltpu
from jax.experimental.pallas import tpu_sc as plsc

N = 10000
E = 160000
F_IN = 128
HID = 256
NC = 40

NCORES = 2
NSUB = 16
EB = E // NSUB
NP = 10240
WR = NP // NSUB
ABATCH = 400
G = 80

_mesh = plsc.VectorSubcoreMesh(core_axis_name="c", subcore_axis_name="s",
                               num_cores=NCORES, num_subcores=NSUB)


def _mm_chunks_body(x_ref, w_ref, o_ref):
    o_ref[0] = jnp.dot(x_ref[...], w_ref[0],
                       preferred_element_type=jnp.float32)


def _mm_chunks(x, wc, chunks, cw, bn=1000):
    f = x.shape[1]
    nb = N // bn
    return pl.pallas_call(
        _mm_chunks_body,
        grid=(nb, chunks),
        in_specs=[pl.BlockSpec((bn, f), lambda i, c: (i, 0)),
                  pl.BlockSpec((1, f, cw), lambda i, c: (c, 0, 0))],
        out_specs=pl.BlockSpec((1, bn, cw), lambda i, c: (c, i, 0)),
        out_shape=jax.ShapeDtypeStruct((chunks, N, cw), jnp.float32),
    )(x, wc)


def _attn_body(x_ref, w_ref, ams_ref, amd_ref, oas_ref, oad_ref):
    x = x_ref[...]
    was = jnp.dot(w_ref[0], ams_ref[0], preferred_element_type=jnp.float32)
    wad = jnp.dot(w_ref[0], amd_ref[0], preferred_element_type=jnp.float32)
    oas_ref[0] = jnp.dot(x, was, preferred_element_type=jnp.float32)
    oad_ref[0] = jnp.dot(x, wad, preferred_element_type=jnp.float32)


def _attn(x, wh, amh_s, amh_d, hd, c, bn=1000):
    f = x.shape[1]
    hdh = hd // 2
    nb = N // bn
    out = jax.ShapeDtypeStruct((2, N, hdh), jnp.float32)
    return pl.pallas_call(
        _attn_body,
        grid=(nb, 2),
        in_specs=[pl.BlockSpec((bn, f), lambda i, s: (i, 0)),
                  pl.BlockSpec((1, f, hdh * c), lambda i, s: (s, 0, 0)),
                  pl.BlockSpec((1, hdh * c, hdh), lambda i, s: (s, 0, 0)),
                  pl.BlockSpec((1, hdh * c, hdh), lambda i, s: (s, 0, 0))],
        out_specs=[pl.BlockSpec((1, bn, hdh), lambda i, s: (s, i, 0)),
                   pl.BlockSpec((1, bn, hdh), lambda i, s: (s, i, 0))],
        out_shape=[out, out],
    )(x, wh, amh_s, amh_d)


def _combine_body(m_ref, x_ref, lw_ref, b_ref, lb_ref, o_ref):
    r = (m_ref[0] + b_ref[...][None, :] + lb_ref[...][None, :]
         + jnp.dot(x_ref[...], lw_ref[...], preferred_element_type=jnp.float32))
    o_ref[...] = jnp.where(r > 0, r, jnp.exp(r) - 1.0)


def _combine(msg, x, lw, b, lb, chunks, cw, bn=1000):
    f = x.shape[1]
    nb = N // bn
    return pl.pallas_call(
        _combine_body,
        grid=(nb, chunks),
        in_specs=[pl.BlockSpec((1, bn, cw), lambda i, c: (c, i, 0)),
                  pl.BlockSpec((bn, f), lambda i, c: (i, 0)),
                  pl.BlockSpec((f, cw), lambda i, c: (0, c)),
                  pl.BlockSpec((cw,), lambda i, c: (c,)),
                  pl.BlockSpec((cw,), lambda i, c: (c,))],
        out_specs=pl.BlockSpec((bn, cw), lambda i, c: (i, c)),
        out_shape=jax.ShapeDtypeStruct((N, chunks * cw), jnp.float32),
    )(msg.reshape(chunks, NP, cw), x, lw, b, lb)


def _final_body(m_ref, x_ref, lw_ref, b_ref, lb_ref, o_ref):
    m = jnp.sum(m_ref[...], axis=0) * (1.0 / 6.0)
    o_ref[...] = (m + b_ref[...][None, :] + lb_ref[...][None, :]
                  + jnp.dot(x_ref[...], lw_ref[...],
                            preferred_element_type=jnp.float32))


def _final(msg, x, lw, b, lb, bn=1000):
    f = x.shape[1]
    nb = N // bn
    return pl.pallas_call(
        _final_body,
        grid=(nb,),
        in_specs=[pl.BlockSpec((6, bn, 64), lambda i: (0, i, 0)),
                  pl.BlockSpec((bn, f), lambda i: (i, 0)),
                  pl.BlockSpec((f, 64), lambda i: (0, 0)),
                  pl.BlockSpec((64,), lambda i: (0,)),
                  pl.BlockSpec((64,), lambda i: (0,))],
        out_specs=pl.BlockSpec((bn, 64), lambda i: (i, 0)),
        out_shape=jax.ShapeDtypeStruct((N, 64), jnp.float32),
    )(msg.reshape(6, NP, 64), x, lw, b, lb)


def _alpha_body(src_hbm, dst_hbm, as_hbm, ad_hbm, alpha_hbm,
                asv, adv, sv, dv, dvo, exv, denr, stage, den_sp, *, hdh):
    cax = lax.axis_index("c")
    tid = lax.axis_index("s")
    e0 = tid * EB
    nbatch = EB // ABATCH
    nh = N * hdh

    pltpu.sync_copy(as_hbm.at[pl.ds(cax * nh, nh)], asv)
    pltpu.sync_copy(ad_hbm.at[pl.ds(cax * nh, nh)], adv)
    pltpu.sync_copy(src_hbm.at[pl.ds(e0, EB)], sv)
    pltpu.sync_copy(dst_hbm.at[pl.ds(e0, EB)], dv)

    zero16 = jnp.zeros((16,), jnp.float32)

    def zb(i, _):
        exv[i, pl.ds(0, 16)] = zero16
        return 0
    lax.fori_loop(0, ABATCH, zb, 0)
    pltpu.sync_copy(exv, den_sp.at[pl.ds(tid * WR, ABATCH)])
    pltpu.sync_copy(exv.at[pl.ds(0, WR - ABATCH)],
                    den_sp.at[pl.ds(tid * WR + ABATCH, WR - ABATCH)])
    plsc.subcore_barrier()

    def _ex16(b, i, hsplat):
        j = b * ABATCH + i * 16
        s16 = sv[pl.ds(j, 16)]
        d16 = dv[pl.ds(j, 16)]
        av = plsc.load_gather(asv, [s16 * hdh + hsplat])
        bv = plsc.load_gather(adv, [d16 * hdh + hsplat])
        e16 = av + bv
        e16 = jnp.where(e16 >= 0, e16, 0.2 * e16)
        return jnp.exp(e16)

    def fill_dvo(b):
        def ob(i, _):
            dvo[pl.ds(i * 16, 16)] = dv[pl.ds(b * ABATCH + i * 16, 16)]
            return 0
        lax.fori_loop(0, ABATCH // 16, ob, 0)

    def p1b(b, _):
        fill_dvo(b)
        for h in range(hdh):
            hsplat = jnp.full((16,), h, jnp.int32)

            def p1(i, _, b=b, hsplat=hsplat):
                ex16 = _ex16(b, i, hsplat)
                plsc.store_scatter(
                    exv, [lax.iota(jnp.int32, 16) + i * 16, hsplat], ex16)
                return 0
            lax.fori_loop(0, ABATCH // 16, p1, 0)
        pltpu.sync_copy(exv, den_sp.at[dvo], add=True)
        return 0
    lax.fori_loop(0, nbatch, p1b, 0)
    plsc.subcore_barrier()

    def p2b(b, _):
        fill_dvo(b)
        pltpu.sync_copy(den_sp.at[dvo], denr)
        for h in range(hdh):
            hsplat = jnp.full((16,), h, jnp.int32)

            def p2(i, _, b=b, hsplat=hsplat):
                ex16 = _ex16(b, i, hsplat)
                e16i = lax.iota(jnp.int32, 16) + i * 16
                den16 = plsc.load_gather(denr, [e16i, hsplat])
                stage[pl.ds(i * 16, 16)] = ex16 / (den16 + 1e-16)
                return 0
            lax.fori_loop(0, ABATCH // 16, p2, 0)
            pltpu.sync_copy(
                stage,
                alpha_hbm.at[pl.ds((cax * hdh + h) * E + e0 + b * ABATCH,
                                   ABATCH)])
        return 0
    lax.fori_loop(0, nbatch, p2b, 0)


def _alpha(src, dst, ass, ads, hd):
    hdh = hd // 2
    body = functools.partial(_alpha_body, hdh=hdh)
    return pl.kernel(
        body,
        out_type=jax.ShapeDtypeStruct((hd * E,), jnp.float32),
        mesh=_mesh,
        compiler_params=pltpu.CompilerParams(
            needs_layout_passes=False, use_tc_tiling_on_sc=False),
        scratch_types=[
            pltpu.VMEM((N * hdh,), jnp.float32),
            pltpu.VMEM((N * hdh,), jnp.float32),
            pltpu.VMEM((EB,), jnp.int32),
            pltpu.VMEM((EB,), jnp.int32),
            pltpu.VMEM((ABATCH,), jnp.int32),
            pltpu.VMEM((ABATCH, 16), jnp.float32),
            pltpu.VMEM((ABATCH, 16), jnp.float32),
            pltpu.VMEM((ABATCH,), jnp.float32),
            pltpu.MemorySpace.VMEM_SHARED((NP, 16), jnp.float32),
        ],
    )(src, dst, ass, ads)


def _msg_body(src_hbm, dst_hbm, alpha_hbm, h_hbm, out_hbm,
              sv, dv, svo_a, svo_b, dvo, av_a, av_b, rows_a, rows_b,
              sem_a, sem_b, acc_sp, *, ch, cw, hpc):
    cax = lax.axis_index("c")
    tid = lax.axis_index("s")
    e0 = tid * EB
    chh = ch // 2
    nv = cw // 16
    nb = EB // G

    pltpu.sync_copy(src_hbm.at[pl.ds(e0, EB)], sv)
    pltpu.sync_copy(dst_hbm.at[pl.ds(e0, EB)], dv)

    zero16 = jnp.zeros((16,), jnp.float32)

    for k in range(chh):
        c = cax * chh + k
        head = c // hpc

        def zb(i, _):
            r = i // nv
            v = i % nv
            rows_a[r, pl.ds(v * 16, 16)] = zero16
            return 0
        lax.fori_loop(0, G * nv, zb, 0)
        for z in range(WR // G):
            pltpu.sync_copy(rows_a, acc_sp.at[pl.ds(tid * WR + z * G, G)])
        plsc.subcore_barrier()

        def fill_svo(b, svo, c=c):
            def obody(i, _):
                svo[pl.ds(i * 16, 16)] = (
                    sv[pl.ds(b * G + i * 16, 16)] + c * N)
                return 0
            lax.fori_loop(0, G // 16, obody, 0)

        def start(b, svo, rows, av, sem, head=head):
            pltpu.async_copy(h_hbm.at[svo], rows, sem)
            pltpu.async_copy(
                alpha_hbm.at[pl.ds(head * E + e0 + b * G, G)], av, sem)

        def drain(rows, av, sem):
            pltpu.make_async_copy(h_hbm.at[pl.ds(0, G)], rows, sem).wait()
            pltpu.make_async_copy(
                alpha_hbm.at[pl.ds(0, G)], av, sem).wait()

        def process(b, rows, av):
            def obody(i, _):
                dvo[pl.ds(i * 16, 16)] = dv[pl.ds(b * G + i * 16, 16)]
                return 0
            lax.fori_loop(0, G // 16, obody, 0)

            def sbody(q, _):
                a16 = av[pl.ds(q * 16, 16)]
                for u in range(16):
                    j = q * 16 + u
                    a = a16[u]
                    for v in range(nv):
                        rows[j, pl.ds(v * 16, 16)] = (
                            rows[j, pl.ds(v * 16, 16)] * a)
                return 0
            lax.fori_loop(0, G // 16, sbody, 0)

            pltpu.sync_copy(rows, acc_sp.at[dvo], add=True)

        fill_svo(0, svo_a)
        start(0, svo_a, rows_a, av_a, sem_a)

        def pair(j, _):
            b0 = 2 * j
            fill_svo(b0 + 1, svo_b)
            start(b0 + 1, svo_b, rows_b, av_b, sem_b)
            drain(rows_a, av_a, sem_a)
            process(b0, rows_a, av_a)
            fill_svo(b0 + 2, svo_a)
            start(b0 + 2, svo_a, rows_a, av_a, sem_a)
            drain(rows_b, av_b, sem_b)
            process(b0 + 1, rows_b, av_b)
            return 0
        lax.fori_loop(0, (nb - 1) // 2, pair, 0)

        drain(rows_a, av_a, sem_a)
        process(nb - 1, rows_a, av_a)
        plsc.subcore_barrier()

        pltpu.sync_copy(acc_sp.at[pl.ds(tid * WR, WR)],
                        out_hbm.at[pl.ds(c * NP + tid * WR, WR)])
        plsc.subcore_barrier()


def _msg(src, dst, alpha_t, hflat, ch, cw, hpc):
    body = functools.partial(_msg_body, ch=ch, cw=cw, hpc=hpc)
    return pl.kernel(
        body,
        out_type=jax.ShapeDtypeStruct((ch * NP, cw), jnp.float32),
        mesh=_mesh,
        compiler_params=pltpu.CompilerParams(
            needs_layout_passes=False, use_tc_tiling_on_sc=False),
        scratch_types=[
            pltpu.VMEM((EB,), jnp.int32),
            pltpu.VMEM((EB,), jnp.int32),
            pltpu.VMEM((G,), jnp.int32),
            pltpu.VMEM((G,), jnp.int32),
            pltpu.VMEM((G,), jnp.int32),
            pltpu.VMEM((G,), jnp.float32),
            pltpu.VMEM((G,), jnp.float32),
            pltpu.VMEM((G, cw), jnp.float32),
            pltpu.VMEM((G, cw), jnp.float32),
            pltpu.SemaphoreType.DMA,
            pltpu.SemaphoreType.DMA,
            pltpu.MemorySpace.VMEM_SHARED((NP, cw), jnp.float32),
        ],
    )(src, dst, alpha_t, hflat)


def _block_diag_halves(a, c):
    hd = a.shape[0]
    hdh = hd // 2
    mask = jnp.repeat(jnp.eye(hdh, dtype=jnp.float32), c, axis=0)
    halves = [mask * a[s * hdh:(s + 1) * hdh].reshape(hdh * c, 1)
              for s in range(2)]
    return jnp.stack(halves)


def _gat_layer(x, src, dst, w, a_s, a_d, hd, chunks, cw, hpc):
    f = x.shape[1]
    c = (chunks * cw) // hd
    wc = w.reshape(f, chunks, cw).transpose(1, 0, 2)
    wh = w.reshape(f, 2, (hd // 2) * c).transpose(1, 0, 2)
    h = _mm_chunks(x, wc, chunks, cw)
    ass, ads = _attn(x, wh, _block_diag_halves(a_s, c),
                     _block_diag_halves(a_d, c), hd, c)
    alpha_t = _alpha(src, dst, ass.reshape(2 * N * (hd // 2)),
                     ads.reshape(2 * N * (hd // 2)), hd)
    msg = _msg(src, dst, alpha_t, h.reshape(chunks * N, cw), chunks, cw, hpc)
    return msg, alpha_t


def kernel(x, edge_index, W1, a1s, a1d, b1, L1W, L1b,
           W2, a2s, a2d, b2, L2W, L2b, W3, a3s, a3d, b3, L3W, L3b):
    src = edge_index[0]
    dst = edge_index[1]

    m1, alpha1_t = _gat_layer(x, src, dst, W1, a1s, a1d,
                              hd=4, chunks=8, cw=128, hpc=2)
    x1 = _combine(m1, x, L1W, b1, L1b, chunks=8, cw=128)

    m2, _ = _gat_layer(x1, src, dst, W2, a2s, a2d,
                       hd=4, chunks=8, cw=128, hpc=2)
    x2 = _combine(m2, x1, L2W, b2, L2b, chunks=8, cw=128)

    w3p = jnp.pad(W3.reshape(4 * HID, 6, NC), ((0, 0), (0, 0), (0, 64 - NC))
                  ).reshape(4 * HID, 6 * 64)
    a3sp = jnp.pad(a3s, ((0, 0), (0, 64 - NC)))
    a3dp = jnp.pad(a3d, ((0, 0), (0, 64 - NC)))
    l3wp = jnp.pad(L3W, ((0, 0), (0, 64 - NC)))
    b3p = jnp.pad(b3, (0, 64 - NC))
    l3bp = jnp.pad(L3b, (0, 64 - NC))
    m3, _ = _gat_layer(x2, src, dst, w3p, a3sp, a3dp,
                       hd=6, chunks=6, cw=64, hpc=1)
    outp = _final(m3, x2, l3wp, b3p, l3bp)

    return outp[:, :NC], alpha1_t.reshape(4, E).T
